# baseline (speedup 1.0000x reference)
import functools

import jax
import jax.numpy as jnp
from jax import lax
from jax.experimental import pallas as pl
from jax.experimental.pallas import tpu as pltpu
from jax.experimental.pallas import tpu_sc as plsc

_VOCAB = 100000
_DIMS = 16
_BATCH = 1024


def _make_sc_gather(V, D, B):
  info = plsc.get_sparse_core_info()
  NC, NS = info.num_cores, info.num_subcores
  NW = NC * NS
  assert B % (8 * NW) == 0
  b_per_w = B // NW
  mesh = plsc.VectorSubcoreMesh(core_axis_name="c", subcore_axis_name="s")

  @functools.partial(
      pl.kernel,
      mesh=mesh,
      out_type=jax.ShapeDtypeStruct((B, D), jnp.float32),
      scratch_types=[
          pltpu.VMEM((b_per_w,), jnp.int32),
          pltpu.VMEM((b_per_w, D), jnp.float32),
          pltpu.SemaphoreType.DMA,
      ],
      compiler_params=pltpu.CompilerParams(use_tc_tiling_on_sc=False),
  )
  def gather_kernel(table_hbm, idx_hbm, out_hbm, idx_v, rows_v, sem):
    wid = lax.axis_index("s") * NC + lax.axis_index("c")
    base = wid * b_per_w
    pltpu.sync_copy(idx_hbm.at[pl.ds(base, b_per_w)], idx_v)
    pltpu.async_copy(table_hbm.at[idx_v], rows_v, sem).wait()
    pltpu.sync_copy(rows_v, out_hbm.at[pl.ds(base, b_per_w)])

  return gather_kernel


def _proj_body(emb_ref, w_ref, b_ref, out_ref):
  out_ref[...] = (
      lax.dot_general(
          emb_ref[...],
          w_ref[...],
          dimension_numbers=(((1,), (1,)), ((), ())),
          preferred_element_type=jnp.float32,
      )
      + b_ref[...]
  )


def _projection(embeds, W, b2d, tv):
  B, D = embeds.shape
  V = W.shape[0]
  grid = (pl.cdiv(V, tv),)
  return pl.pallas_call(
      _proj_body,
      grid=grid,
      in_specs=[
          pl.BlockSpec((B, D), lambda j: (0, 0)),
          pl.BlockSpec((tv, D), lambda j: (j, 0)),
          pl.BlockSpec((1, tv), lambda j: (0, j)),
      ],
      out_specs=pl.BlockSpec((B, tv), lambda j: (0, j)),
      out_shape=jax.ShapeDtypeStruct((B, V), jnp.float32),
      compiler_params=pltpu.CompilerParams(
          dimension_semantics=("arbitrary",),
      ),
  )(embeds, W, b2d)


@jax.jit
def kernel(inputs, emb_table, W, b):
  gather = _make_sc_gather(_VOCAB, _DIMS, _BATCH)
  embeds = gather(emb_table, inputs.astype(jnp.int32))
  return _projection(embeds, W, b.reshape(1, _VOCAB), 2048)
